# baseline (speedup 1.0000x reference)
import functools

import jax
import jax.numpy as jnp
from jax import lax
from jax.experimental import pallas as pl
from jax.experimental.pallas import tpu as pltpu
from jax.experimental.pallas import tpu_sc as plsc

N = 10000
E = 160000
H = 128
B = 64
NT = 32
RPT = 320
NP = NT * RPT
NEG = -3.0e38
CE = 2000
CQ = 512
ECAP = 163840
AGGW = 41216
DUMMY_OFF = RPT * 128
BLK = 256
NB = NP // BLK
ACCR = 72


def _wid():
    return lax.axis_index("s") * 2 + lax.axis_index("c")


def _k1_body(v0_ref, v1_ref, w0_ref, w1_ref, base_ref, eWx_ref, edgeW_ref,
             eWe_ref, edgeb_ref, eb_ref, x_ref, xe_ref, we8_ref, ball8_ref):
    x = base_ref[...] + v0_ref[...] * w0_ref[...] + v1_ref[...] * w1_ref[...]
    x_ref[...] = x
    xe_ref[...] = jnp.dot(x, eWx_ref[...], preferred_element_type=jnp.float32)
    we = jnp.dot(edgeW_ref[...], eWe_ref[...], preferred_element_type=jnp.float32)
    we8_ref[...] = jnp.broadcast_to(we, (8, H))
    ball = jnp.dot(edgeb_ref[...], eWe_ref[...],
                   preferred_element_type=jnp.float32) + eb_ref[...]
    ball8_ref[...] = jnp.broadcast_to(ball, (8, H))


def _k1(v0, v1, w0, w1, base, eWx, edgeW, eWe, edgeb, eb):
    full = lambda r, c: pl.BlockSpec((r, c), lambda i: (0, 0))
    blk = pl.BlockSpec((BLK, 1), lambda i: (i, 0))
    out = pl.BlockSpec((BLK, H), lambda i: (i, 0))
    return pl.pallas_call(
        _k1_body,
        grid=(NB,),
        in_specs=[blk, blk, full(1, H), full(1, H), full(1, H), full(H, H),
                  full(1, H), full(H, H), full(1, H), full(1, H)],
        out_specs=[out, out, full(8, H), full(8, H)],
        out_shape=[jax.ShapeDtypeStruct((NP, H), jnp.float32),
                   jax.ShapeDtypeStruct((NP, H), jnp.float32),
                   jax.ShapeDtypeStruct((8, H), jnp.float32),
                   jax.ShapeDtypeStruct((8, H), jnp.float32)],
    )(v0, v1, w0, w1, base, eWx, edgeW, eWe, edgeb, eb)


def _k2_body(col_hbm, row_hbm, dist_hbm, qoff_hbm, qdist_hbm, qrow_hbm,
             qcnt_hbm, colb, rowb, distb, qoffb, qrowb, qdistb, cntb):
    wid = _wid()
    lo = wid * RPT
    hi = lo + RPT
    qb = wid * ECAP

    def prefill(g, c):
        qoffb[pl.ds(g * 16, 16)] = jnp.full((16,), DUMMY_OFF, jnp.int32)
        qrowb[pl.ds(g * 16, 16)] = jnp.zeros((16,), jnp.int32)
        qdistb[pl.ds(g * 16, 16)] = jnp.zeros((16,), jnp.float32)
        return c

    def chunk(ci, goff):
        pltpu.sync_copy(col_hbm.at[pl.ds(ci * CE, CE)], colb)
        pltpu.sync_copy(row_hbm.at[pl.ds(ci * CE, CE)], rowb)
        pltpu.sync_copy(dist_hbm.at[pl.ds(ci * CE, CE)], distb)
        lax.fori_loop(0, 2048 // 16, prefill, 0)

        def grp(g, off):
            cv = colb[pl.ds(g * 16, 16)]
            m = (cv >= lo) & (cv < hi)
            ov = (cv - lo) << 7
            rv = rowb[pl.ds(g * 16, 16)]
            dv = distb[pl.ds(g * 16, 16)]
            idx = off + plsc.cumsum(m.astype(jnp.int32)) - 1
            plsc.store_scatter(qoffb, [idx], ov, mask=m)
            plsc.store_scatter(qrowb, [idx], rv, mask=m)
            plsc.store_scatter(qdistb, [idx], dv, mask=m)
            cnt = plsc.all_reduce_population_count(m)[0]
            return off + cnt

        off = lax.fori_loop(0, CE // 16, grp, 0)
        off16 = (off + 15) & ~15
        fo = pl.multiple_of(qb + goff, 16)
        pltpu.sync_copy(qoffb, qoff_hbm.at[pl.ds(fo, 2048)])
        pltpu.sync_copy(qrowb, qrow_hbm.at[pl.ds(fo, 2048)])
        pltpu.sync_copy(qdistb, qdist_hbm.at[pl.ds(fo, 2048)])
        return goff + off16

    goff = lax.fori_loop(0, E // CE, chunk, 0)
    lax.fori_loop(0, 2048 // 16, prefill, 0)
    fo = pl.multiple_of(qb + goff, 16)
    pltpu.sync_copy(qoffb.at[pl.ds(0, CQ)], qoff_hbm.at[pl.ds(fo, CQ)])
    pltpu.sync_copy(qrowb.at[pl.ds(0, CQ)], qrow_hbm.at[pl.ds(fo, CQ)])
    pltpu.sync_copy(qdistb.at[pl.ds(0, CQ)], qdist_hbm.at[pl.ds(fo, CQ)])
    cntb[...] = jnp.full((16,), goff, jnp.int32)
    pltpu.sync_copy(cntb, qcnt_hbm.at[pl.ds(pl.multiple_of(wid * 16, 16), 16)])


def _k3_body(xe_hbm, qoff_hbm, qdist_hbm, qrow_hbm, qcnt_hbm, we_hbm, out_hbm,
             aggb, qoffb, qrowb, qdistb, rowsb, web, cntb, sem):
    wid = _wid()

    def ini(g, c):
        aggb[pl.ds(g * 16, 16)] = jnp.full((16,), NEG, jnp.float32)
        return c

    lax.fori_loop(0, AGGW // 16, ini, 0)
    pltpu.sync_copy(we_hbm, web)
    pltpu.sync_copy(qcnt_hbm.at[pl.ds(pl.multiple_of(wid * 16, 16), 16)], cntb)
    cnt = cntb[...][0]
    wvecs = [web[pl.ds(k * 16, 16)] for k in range(8)]

    def chunk(ci, c):
        base = pl.multiple_of(wid * ECAP + ci * CQ, CQ)
        pltpu.sync_copy(qrow_hbm.at[pl.ds(base, CQ)], qrowb)
        pltpu.sync_copy(qoff_hbm.at[pl.ds(base, CQ)], qoffb)
        pltpu.sync_copy(qdist_hbm.at[pl.ds(base, CQ)], qdistb)
        copies = []
        for j in range(CQ // 128):
            cp = pltpu.make_async_copy(
                xe_hbm.at[qrowb.at[pl.ds(j * 128, 128)]],
                rowsb.at[pl.ds(j * 128, 128)], sem)
            cp.start()
            copies.append(cp)
        for cp in copies:
            cp.wait()

        def grp16(g, cc):
            qo = qoffb[pl.ds(g * 16, 16)]
            qd = qdistb[pl.ds(g * 16, 16)]
            for l in range(16):
                off = qo[l]
                dv = jnp.full((16,), qd[l], jnp.float32)
                e = g * 16 + l
                for k in range(8):
                    z = rowsb[e, pl.ds(k * 16, 16)] + dv * wvecs[k]
                    a = aggb[pl.ds(off + k * 16, 16)]
                    aggb[pl.ds(off + k * 16, 16)] = jnp.maximum(a, z)
            return cc

        lax.fori_loop(0, CQ // 16, grp16, 0)
        return c

    lax.fori_loop(0, (cnt + CQ - 1) // CQ, chunk, 0)
    pltpu.sync_copy(aggb.at[pl.ds(0, RPT * H)],
                    out_hbm.at[pl.ds(pl.multiple_of(wid * RPT * H, 128), RPT * H)])


def _k5_body(x_hbm, batch_hbm, parts_hbm, xb, bb, accb):
    wid = _wid()
    pltpu.sync_copy(x_hbm.at[pl.ds(pl.multiple_of(wid * RPT * H, 128), RPT * H)], xb)
    pltpu.sync_copy(batch_hbm.at[pl.ds(pl.multiple_of(wid * RPT, 64), RPT)], bb)

    def ini(g, c):
        accb[pl.ds(g * 16, 16)] = jnp.full((16,), NEG, jnp.float32)
        return c

    lax.fori_loop(0, ACCR * H // 16, ini, 0)

    def rowf(g, c):
        bv = bb[pl.ds(g * 16, 16)]
        for l in range(16):
            off = bv[l] << 7
            r = g * 16 + l
            for k in range(8):
                a = accb[pl.ds(off + k * 16, 16)]
                v = xb[pl.ds(r * H + k * 16, 16)]
                accb[pl.ds(off + k * 16, 16)] = jnp.maximum(a, v)
        return c

    lax.fori_loop(0, RPT // 16, rowf, 0)
    pltpu.sync_copy(accb, parts_hbm.at[pl.ds(pl.multiple_of(wid * ACCR * H, 128), ACCR * H)])


@functools.lru_cache(maxsize=None)
def _sc_kernels():
    mesh = plsc.VectorSubcoreMesh(core_axis_name="c", subcore_axis_name="s")
    cp = pltpu.CompilerParams(needs_layout_passes=False)
    k2 = pl.kernel(
        _k2_body,
        mesh=mesh,
        compiler_params=cp,
        out_type=[jax.ShapeDtypeStruct((NT * ECAP,), jnp.int32),
                  jax.ShapeDtypeStruct((NT * ECAP,), jnp.float32),
                  jax.ShapeDtypeStruct((NT * ECAP,), jnp.int32),
                  jax.ShapeDtypeStruct((NT * 16,), jnp.int32)],
        scratch_types=[pltpu.VMEM((CE,), jnp.int32),
                       pltpu.VMEM((CE,), jnp.int32),
                       pltpu.VMEM((CE,), jnp.float32),
                       pltpu.VMEM((2048,), jnp.int32),
                       pltpu.VMEM((2048,), jnp.int32),
                       pltpu.VMEM((2048,), jnp.float32),
                       pltpu.VMEM((16,), jnp.int32)],
    )
    k3 = pl.kernel(
        _k3_body,
        mesh=mesh,
        compiler_params=cp,
        out_type=jax.ShapeDtypeStruct((NP * H,), jnp.float32),
        scratch_types=[pltpu.VMEM((AGGW,), jnp.float32),
                       pltpu.VMEM((CQ,), jnp.int32),
                       pltpu.VMEM((CQ,), jnp.int32),
                       pltpu.VMEM((CQ,), jnp.float32),
                       pltpu.VMEM((CQ, H), jnp.float32),
                       pltpu.VMEM((H,), jnp.float32),
                       pltpu.VMEM((16,), jnp.int32),
                       pltpu.SemaphoreType.DMA],
    )
    k5 = pl.kernel(
        _k5_body,
        mesh=mesh,
        compiler_params=cp,
        out_type=jax.ShapeDtypeStruct((NT * ACCR * H,), jnp.float32),
        scratch_types=[pltpu.VMEM((RPT * H,), jnp.float32),
                       pltpu.VMEM((RPT,), jnp.int32),
                       pltpu.VMEM((ACCR * H,), jnp.float32)],
    )
    return k2, k3, k5


def _k4_common(agg_ref, x_ref, b_ref, ub4_ref, Wa_ref, Wx_ref, nb1_ref,
               nW2_ref, nb2_ref, ball_ref):
    agg = jnp.maximum(agg_ref[...] + ball_ref[0:1, :], 0.0)
    bvec = b_ref[0, 0, :]
    oneh = (bvec[:, None] == lax.broadcasted_iota(jnp.int32, (BLK, B), 1)
            ).astype(jnp.float32)
    h1 = jnp.dot(agg, Wa_ref[...], preferred_element_type=jnp.float32)
    h1 += jnp.dot(x_ref[...], Wx_ref[...], preferred_element_type=jnp.float32)
    h1 += jnp.dot(oneh, ub4_ref[...], preferred_element_type=jnp.float32)
    h1 = jnp.maximum(h1 + nb1_ref[...], 0.0)
    return x_ref[...] + jnp.dot(h1, nW2_ref[...],
                                preferred_element_type=jnp.float32) + nb2_ref[...]


def _k4_body_a(agg_ref, x_ref, b_ref, ub4_ref, Wa_ref, Wx_ref, nb1_ref,
               nW2_ref, nb2_ref, ball_ref, Wex_ref, bias2_ref, xn_ref, ex_ref):
    xn = _k4_common(agg_ref, x_ref, b_ref, ub4_ref, Wa_ref, Wx_ref, nb1_ref,
                    nW2_ref, nb2_ref, ball_ref)
    xn_ref[...] = xn
    ex_ref[...] = jnp.dot(xn, Wex_ref[...], preferred_element_type=jnp.float32)


def _k4_body_b(agg_ref, x_ref, b_ref, ub4_ref, Wa_ref, Wx_ref, nb1_ref,
               nW2_ref, nb2_ref, ball_ref, Wex_ref, bias2_ref, xn_ref, ex_ref):
    xn = _k4_common(agg_ref, x_ref, b_ref, ub4_ref, Wa_ref, Wx_ref, nb1_ref,
                    nW2_ref, nb2_ref, ball_ref)
    xn_ref[...] = xn
    ex_ref[...] = jnp.dot(xn, Wex_ref[...],
                          preferred_element_type=jnp.float32) + bias2_ref[...]


def _k4(last, agg, x, batch3, ub4, Wa, Wx, nb1, nW2, nb2, ball8, Wex, bias2):
    full = lambda r, c: pl.BlockSpec((r, c), lambda i: (0, 0))
    out = pl.BlockSpec((BLK, H), lambda i: (i, 0))
    body = _k4_body_b if last else _k4_body_a
    return pl.pallas_call(
        body,
        grid=(NB,),
        in_specs=[out, out, pl.BlockSpec((1, 1, BLK), lambda i: (i, 0, 0)),
                  full(B, 4 * H), full(H, 4 * H), full(H, 4 * H),
                  full(1, 4 * H), full(4 * H, H), full(1, H), full(8, H),
                  full(H, H), full(1, H)],
        out_specs=[out, out],
        out_shape=[jax.ShapeDtypeStruct((NP, H), jnp.float32),
                   jax.ShapeDtypeStruct((NP, H), jnp.float32)],
    )(agg, x, batch3, ub4, Wa, Wx, nb1, nW2, nb2, ball8, Wex, bias2)


def _k6_body(parts_ref, u_ref, gW1_ref, gW2_ref, gb_ref, Wih_ref, bsum_ref,
             Wu_ref, u_out_ref, c_out_ref, ub4_ref):
    xraw = jnp.max(parts_ref[:, :B, :], axis=0)
    xu = jnp.where(xraw <= NEG * 0.5, 0.0, xraw)
    u = u_ref[...]
    g = jnp.maximum(jnp.dot(xu, gW1_ref[...], preferred_element_type=jnp.float32)
                    + jnp.dot(u, gW2_ref[...], preferred_element_type=jnp.float32)
                    + gb_ref[...], 0.0)
    um = u + g
    gates = jnp.dot(um, Wih_ref[...],
                    preferred_element_type=jnp.float32) + bsum_ref[...]
    i_g = gates[:, 0:H]
    g_g = gates[:, 2 * H:3 * H]
    o_g = gates[:, 3 * H:4 * H]
    c = jax.nn.sigmoid(i_g) * jnp.tanh(g_g)
    h = jax.nn.sigmoid(o_g) * jnp.tanh(c)
    u_out_ref[...] = h
    c_out_ref[...] = c
    ub4_ref[...] = jnp.dot(h, Wu_ref[...], preferred_element_type=jnp.float32)


def _k6(parts, u, gW1, gW2, gb, Wih, bsum, Wu):
    full = lambda s: pl.BlockSpec(s, lambda: tuple(0 for _ in s))
    return pl.pallas_call(
        _k6_body,
        grid=(),
        in_specs=[full((NT, ACCR, H)), full((B, H)), full((H, H)),
                  full((H, H)), full((1, H)), full((H, 4 * H)),
                  full((1, 4 * H)), full((H, 4 * H))],
        out_specs=[full((B, H)), full((B, H)), full((B, 4 * H))],
        out_shape=[jax.ShapeDtypeStruct((B, H), jnp.float32),
                   jax.ShapeDtypeStruct((B, H), jnp.float32),
                   jax.ShapeDtypeStruct((B, 4 * H), jnp.float32)],
    )(parts, u, gW1, gW2, gb, Wih, bsum, Wu)


def kernel(v0, v1, dist, edge_index, batch, batch_size, wte, in_W0, in_b0,
           in_W1, in_b1, out_W0, out_b0, out_W1, out_b1, edge_W, edge_b, eW,
           eb, nW1, nb1, nW2, nb2, gW, gb, W_ih, b_ih, W_hh, b_hh):
    f32 = jnp.float32
    v0p = jnp.pad(v0, ((0, NP - N), (0, 0)))
    v1p = jnp.pad(v1, ((0, NP - N), (0, 0)))
    base = (wte[0] + wte[1] + in_b0 + in_b1).reshape(1, H)
    w0 = in_W0[0].reshape(1, H)
    w1 = in_W1[0].reshape(1, H)
    eWx = eW[:H]
    eWe = eW[H:]
    edgeW = edge_W.reshape(1, H)
    edgeb = edge_b.reshape(1, H)
    eb2 = eb.reshape(1, H)
    row = edge_index[0]
    col = edge_index[1]
    distf = dist.reshape(E)
    batch_pad = jnp.concatenate([batch.astype(jnp.int32),
                                 jnp.full((NP - N,), B, jnp.int32)])
    batch3 = batch_pad.reshape(NB, 1, BLK)
    Wa, Wx, Wu = nW1[:H], nW1[H:2 * H], nW1[2 * H:]
    nb1r = nb1.reshape(1, 4 * H)
    nb2r = nb2.reshape(1, H)
    gW1, gW2 = gW[:H], gW[H:]
    gbr = gb.reshape(1, H)
    bsum = (b_ih + b_hh).reshape(1, 4 * H)
    W2p = jnp.concatenate([out_W0, out_W1, jnp.zeros((H, H - 2), f32)], axis=1)
    bias2 = jnp.concatenate([out_b0, out_b1,
                             jnp.zeros((H - 2,), f32)]).reshape(1, H)

    k2, k3, k5 = _sc_kernels()

    x, xe, we8, ball8 = _k1(v0p, v1p, w0, w1, base, eWx, edgeW, eWe, edgeb, eb2)
    we_vec = we8[0]

    qoff, qdist, qrow, qcnt = k2(col, row, distf)

    u = jnp.zeros((B, H), f32)
    ub4 = jnp.zeros((B, 4 * H), f32)
    cs = []
    for layer in range(2):
        agg_flat = k3(xe, qoff, qdist, qrow, qcnt, we_vec)
        agg = agg_flat.reshape(NP, H)
        last = layer == 1
        xn, ex = _k4(last, agg, x, batch3, ub4, Wa, Wx, nb1r, nW2, nb2r,
                     ball8, W2p if last else eWx, bias2)
        x = xn
        parts = k5(x.reshape(NP * H), batch_pad)
        u, c, ub4 = _k6(parts.reshape(NT, ACCR, H), u, gW1, gW2, gbr,
                        W_ih, bsum, Wu)
        cs.append(c)
        if not last:
            xe = ex
    out0 = ex[:N, 0:1]
    out1 = ex[:N, 1:2]
    return (out0, out1, cs[0], cs[1])
